# Initial kernel scaffold; baseline (speedup 1.0000x reference)
import jax
import jax.numpy as jnp
from jax.experimental import pallas as pl


def kernel(pred_logits, pred_boxes, target_sizes):
    raise NotImplementedError("write your pallas kernel here")


# speedup vs baseline: 14.0601x; 14.0601x over previous
import jax
import jax.numpy as jnp
from jax.experimental import pallas as pl

_B, _N, _C = 8, 20000, 91
_KROWS = 128
_KOUT = 100


def _rowmax_kernel(x_ref, o_ref):
    o_ref[...] = jnp.max(x_ref[...], axis=2)[:, None, :]


def _toprows_kernel(rm_ref, rows_ref):
    x = rm_ref[...]
    iota = jax.lax.broadcasted_iota(jnp.int32, (_B, _N), 1)
    lane = jax.lax.broadcasted_iota(jnp.int32, (_B, _KROWS), 1)

    def body(i, carry):
        x, acc = carry
        m = jnp.max(x, axis=1, keepdims=True)
        idx = jnp.min(jnp.where(x == m, iota, jnp.int32(_N)), axis=1,
                      keepdims=True)
        acc = jnp.where(lane == i, idx, acc)
        x = jnp.where(iota == idx, -jnp.inf, x)
        return x, acc

    _, acc = jax.lax.fori_loop(
        0, _KROWS, body, (x, jnp.zeros((_B, _KROWS), jnp.int32)))
    rows_ref[...] = acc


def _topk_kernel(cand_ref, scores_ref, q_ref):
    x = cand_ref[...]
    m_lanes = _KROWS * _C
    iota = jax.lax.broadcasted_iota(jnp.int32, (_B, m_lanes), 1)
    lane = jax.lax.broadcasted_iota(jnp.int32, (_B, _KOUT), 1)

    def body(i, carry):
        x, accv, accq = carry
        m = jnp.max(x, axis=1, keepdims=True)
        idx = jnp.min(jnp.where(x == m, iota, jnp.int32(m_lanes)), axis=1,
                      keepdims=True)
        accv = jnp.where(lane == i, m, accv)
        accq = jnp.where(lane == i, idx, accq)
        x = jnp.where(iota == idx, -jnp.inf, x)
        return x, accv, accq

    _, vals, qs = jax.lax.fori_loop(
        0, _KOUT, body,
        (x, jnp.zeros((_B, _KOUT), jnp.float32),
         jnp.zeros((_B, _KOUT), jnp.int32)))
    scores_ref[...] = jax.nn.sigmoid(vals)
    q_ref[...] = qs


def kernel(pred_logits, pred_boxes, target_sizes):
    del target_sizes
    rowmax = pl.pallas_call(
        _rowmax_kernel,
        grid=(_B,),
        in_specs=[pl.BlockSpec((1, _N, _C), lambda b: (b, 0, 0))],
        out_specs=pl.BlockSpec((1, 1, _N), lambda b: (b, 0, 0)),
        out_shape=jax.ShapeDtypeStruct((_B, 1, _N), jnp.float32),
    )(pred_logits).reshape(_B, _N)

    cand_rows = pl.pallas_call(
        _toprows_kernel,
        in_specs=[pl.BlockSpec((_B, _N), lambda: (0, 0))],
        out_specs=pl.BlockSpec((_B, _KROWS), lambda: (0, 0)),
        out_shape=jax.ShapeDtypeStruct((_B, _KROWS), jnp.int32),
    )(rowmax)

    cand_rows = jnp.sort(cand_rows, axis=1)
    cand = jnp.take_along_axis(
        pred_logits, cand_rows[:, :, None], axis=1).reshape(_B, _KROWS * _C)

    scores, q = pl.pallas_call(
        _topk_kernel,
        in_specs=[pl.BlockSpec((_B, _KROWS * _C), lambda: (0, 0))],
        out_specs=[
            pl.BlockSpec((_B, _KOUT), lambda: (0, 0)),
            pl.BlockSpec((_B, _KOUT), lambda: (0, 0)),
        ],
        out_shape=[
            jax.ShapeDtypeStruct((_B, _KOUT), jnp.float32),
            jax.ShapeDtypeStruct((_B, _KOUT), jnp.int32),
        ],
    )(cand)

    labels = q % _C
    topk_rows = jnp.take_along_axis(cand_rows, q // _C, axis=1)
    boxes = jnp.take_along_axis(pred_boxes, topk_rows[:, :, None], axis=1)
    return scores, labels, boxes


# speedup vs baseline: 14.8221x; 1.0542x over previous
import jax
import jax.numpy as jnp
from jax.experimental import pallas as pl

_B, _N, _C = 8, 20000, 91
_G = 16
_NG = _N // _G
_KROWS = 128
_KOUT = 100


def _extract_topk(x, k, width):
    iota = jax.lax.broadcasted_iota(jnp.int32, (_B, width), 1)
    lane = jax.lax.broadcasted_iota(jnp.int32, (_B, k), 1)

    def body(i, carry):
        x, accv, accq = carry
        m = jnp.max(x, axis=1, keepdims=True)
        idx = jnp.min(jnp.where(x == m, iota, jnp.int32(width)), axis=1,
                      keepdims=True)
        accv = jnp.where(lane == i, m, accv)
        accq = jnp.where(lane == i, idx, accq)
        x = jnp.where(iota == idx, -jnp.inf, x)
        return x, accv, accq

    _, vals, qs = jax.lax.fori_loop(
        0, k, body,
        (x, jnp.zeros((_B, k), jnp.float32), jnp.zeros((_B, k), jnp.int32)))
    return vals, qs


def _rowmax_kernel(x_ref, o_ref):
    xt = x_ref[0].T
    o_ref[...] = jnp.max(xt, axis=0)[None, None, :]


def _topgroups_kernel(rm_ref, grp_ref):
    gm = jnp.max(rm_ref[...].reshape(_B, _NG, _G), axis=2)
    _, qs = _extract_topk(gm, _KROWS, _NG)
    grp_ref[...] = qs


def _toprows_kernel(win_ref, q_ref):
    _, qs = _extract_topk(win_ref[...], _KROWS, _KROWS * _G)
    q_ref[...] = qs


def _topk_kernel(cand_ref, scores_ref, q_ref):
    vals, qs = _extract_topk(cand_ref[...], _KOUT, _KROWS * _C)
    scores_ref[...] = jax.nn.sigmoid(vals)
    q_ref[...] = qs


def _full_spec(*shape):
    return pl.BlockSpec(shape, lambda: tuple(0 for _ in shape))


def kernel(pred_logits, pred_boxes, target_sizes):
    del target_sizes
    rowmax = pl.pallas_call(
        _rowmax_kernel,
        grid=(_B,),
        in_specs=[pl.BlockSpec((1, _N, _C), lambda b: (b, 0, 0))],
        out_specs=pl.BlockSpec((1, 1, _N), lambda b: (b, 0, 0)),
        out_shape=jax.ShapeDtypeStruct((_B, 1, _N), jnp.float32),
    )(pred_logits).reshape(_B, _N)

    top_groups = pl.pallas_call(
        _topgroups_kernel,
        in_specs=[_full_spec(_B, _N)],
        out_specs=_full_spec(_B, _KROWS),
        out_shape=jax.ShapeDtypeStruct((_B, _KROWS), jnp.int32),
    )(rowmax)

    top_groups = jnp.sort(top_groups, axis=1)
    win_rows = (top_groups[:, :, None] * _G +
                jnp.arange(_G, dtype=jnp.int32)[None, None, :])
    win_rows = win_rows.reshape(_B, _KROWS * _G)
    windows = jnp.take_along_axis(rowmax, win_rows, axis=1)

    rq = pl.pallas_call(
        _toprows_kernel,
        in_specs=[_full_spec(_B, _KROWS * _G)],
        out_specs=_full_spec(_B, _KROWS),
        out_shape=jax.ShapeDtypeStruct((_B, _KROWS), jnp.int32),
    )(windows)

    cand_rows = jnp.take_along_axis(win_rows, rq, axis=1)
    cand_rows = jnp.sort(cand_rows, axis=1)
    cand = jnp.take_along_axis(
        pred_logits, cand_rows[:, :, None], axis=1).reshape(_B, _KROWS * _C)

    scores, q = pl.pallas_call(
        _topk_kernel,
        in_specs=[_full_spec(_B, _KROWS * _C)],
        out_specs=[_full_spec(_B, _KOUT), _full_spec(_B, _KOUT)],
        out_shape=[
            jax.ShapeDtypeStruct((_B, _KOUT), jnp.float32),
            jax.ShapeDtypeStruct((_B, _KOUT), jnp.int32),
        ],
    )(cand)

    labels = q % _C
    topk_rows = jnp.take_along_axis(cand_rows, q // _C, axis=1)
    boxes = jnp.take_along_axis(pred_boxes, topk_rows[:, :, None], axis=1)
    return scores, labels, boxes
